# baseline (speedup 1.0000x reference)
import jax
import jax.numpy as jnp
from jax import lax
from jax.experimental import pallas as pl
from jax.experimental.pallas import tpu as pltpu
from jax.experimental.pallas import tpu_sc as plsc

_NC, _NS = 2, 16
_NW = _NC * _NS
_CHUNK_ROWS = 32
_NBUF = 3
_SC_ROWS = 3072
_TC_BLK = 512


def _sc_copy_body(emb_hbm, out_hbm, bufs, sems_in, sems_out):
    seq, dim = out_hbm.shape
    rows_per_w = seq // _NW
    n_chunks = rows_per_w // _CHUNK_ROWS
    wid = lax.axis_index("s") * _NC + lax.axis_index("c")
    base = wid * rows_per_w

    in_copies = [None] * n_chunks
    out_copies = [None] * n_chunks

    def start_in(i):
        b = i % _NBUF
        off = base + i * _CHUNK_ROWS
        c = pltpu.make_async_copy(
            emb_hbm.at[pl.ds(off, _CHUNK_ROWS)], bufs.at[b], sems_in.at[b])
        c.start()
        in_copies[i] = c

    def start_out(i):
        b = i % _NBUF
        off = base + i * _CHUNK_ROWS
        c = pltpu.make_async_copy(
            bufs.at[b], out_hbm.at[pl.ds(off, _CHUNK_ROWS)], sems_out.at[b])
        c.start()
        out_copies[i] = c

    for i in range(n_chunks + 1):
        if i < n_chunks:
            if i >= _NBUF:
                out_copies[i - _NBUF].wait()
            start_in(i)
        if i >= 1:
            in_copies[i - 1].wait()
            start_out(i - 1)
    for i in range(max(n_chunks - _NBUF, 0), n_chunks):
        out_copies[i].wait()


def _tc_copy_block(emb_ref, o_ref):
    o_ref[...] = emb_ref[...]


def kernel(x, emb):
    seq, dim = x.shape[1], emb.shape[1]
    mesh = plsc.VectorSubcoreMesh(core_axis_name="c", subcore_axis_name="s")
    sc_part = pl.kernel(
        _sc_copy_body,
        out_type=jax.ShapeDtypeStruct((_SC_ROWS, dim), emb.dtype),
        mesh=mesh,
        scratch_types=[
            pltpu.VMEM((_NBUF, _CHUNK_ROWS, dim), emb.dtype),
            pltpu.SemaphoreType.DMA((_NBUF,)),
            pltpu.SemaphoreType.DMA((_NBUF,)),
        ],
    )(emb)
    tc_rows = seq - _SC_ROWS
    tc_part = pl.pallas_call(
        _tc_copy_block,
        grid=(tc_rows // _TC_BLK,),
        in_specs=[pl.BlockSpec((_TC_BLK, dim),
                               lambda i: (i + _SC_ROWS // _TC_BLK, 0))],
        out_specs=pl.BlockSpec((_TC_BLK, dim), lambda i: (i, 0)),
        out_shape=jax.ShapeDtypeStruct((tc_rows, dim), emb.dtype),
        compiler_params=pltpu.CompilerParams(
            dimension_semantics=("arbitrary",),
        ),
    )(emb)
    return jnp.concatenate([sc_part, tc_part], axis=0)


# speedup vs baseline: 1.4137x; 1.4137x over previous
import jax
import jax.numpy as jnp
from jax import lax
from jax.experimental import pallas as pl
from jax.experimental.pallas import tpu as pltpu
from jax.experimental.pallas import tpu_sc as plsc

_NC, _NS = 2, 16
_NW = _NC * _NS
_CHUNK_ROWS = 16
_NBUF = 7


def _sc_copy_body(emb_hbm, out_hbm, bufs, sems_in, sems_out):
    seq, dim = out_hbm.shape
    rows_per_w = seq // _NW
    n_chunks = rows_per_w // _CHUNK_ROWS
    wid = lax.axis_index("s") * _NC + lax.axis_index("c")
    base = wid * rows_per_w

    in_copies = [None] * n_chunks
    out_copies = [None] * n_chunks

    def start_in(i):
        b = i % _NBUF
        off = base + i * _CHUNK_ROWS
        c = pltpu.make_async_copy(
            emb_hbm.at[pl.ds(off, _CHUNK_ROWS)], bufs.at[b], sems_in.at[b])
        c.start()
        in_copies[i] = c

    def start_out(i):
        b = i % _NBUF
        off = base + i * _CHUNK_ROWS
        c = pltpu.make_async_copy(
            bufs.at[b], out_hbm.at[pl.ds(off, _CHUNK_ROWS)], sems_out.at[b])
        c.start()
        out_copies[i] = c

    for i in range(n_chunks + 1):
        if i < n_chunks:
            if i >= _NBUF:
                out_copies[i - _NBUF].wait()
            start_in(i)
        if i >= 1:
            in_copies[i - 1].wait()
            start_out(i - 1)
    for i in range(max(n_chunks - _NBUF, 0), n_chunks):
        out_copies[i].wait()


def kernel(x, emb):
    seq, dim = x.shape[1], emb.shape[1]
    mesh = plsc.VectorSubcoreMesh(core_axis_name="c", subcore_axis_name="s")
    k = pl.kernel(
        _sc_copy_body,
        out_type=jax.ShapeDtypeStruct((seq, dim), emb.dtype),
        mesh=mesh,
        scratch_types=[
            pltpu.VMEM((_NBUF, _CHUNK_ROWS, dim), emb.dtype),
            pltpu.SemaphoreType.DMA((_NBUF,)),
            pltpu.SemaphoreType.DMA((_NBUF,)),
        ],
    )
    return k(emb)


# speedup vs baseline: 1.4334x; 1.0139x over previous
import jax
import jax.numpy as jnp
from jax import lax
from jax.experimental import pallas as pl
from jax.experimental.pallas import tpu as pltpu
from jax.experimental.pallas import tpu_sc as plsc

_NC, _NS = 2, 16
_NW = _NC * _NS
_CHUNK_ROWS = 16
_NBUF = 3
_NBUF_SH = 2


def _sc_copy_body(emb_hbm, out_hbm, bufs, shbufs, sems_in, sems_out,
                  sems_shin, sems_shout):
    seq, dim = out_hbm.shape
    rows_per_w = seq // _NW
    n_chunks = rows_per_w // _CHUNK_ROWS
    wid = lax.axis_index("s") * _NC + lax.axis_index("c")
    sid = lax.axis_index("s")
    base = wid * rows_per_w

    in_copies = [None] * n_chunks
    out_copies = [None] * n_chunks

    def start_in(i):
        off = base + i * _CHUNK_ROWS
        if i % 2 == 0:
            b = (i // 2) % _NBUF
            c = pltpu.make_async_copy(
                emb_hbm.at[pl.ds(off, _CHUNK_ROWS)], bufs.at[b],
                sems_in.at[b])
        else:
            b = (i // 2) % _NBUF_SH
            c = pltpu.make_async_copy(
                emb_hbm.at[pl.ds(off, _CHUNK_ROWS)], shbufs.at[sid, b],
                sems_shin.at[b])
        c.start()
        in_copies[i] = c

    def start_out(i):
        off = base + i * _CHUNK_ROWS
        if i % 2 == 0:
            b = (i // 2) % _NBUF
            c = pltpu.make_async_copy(
                bufs.at[b], out_hbm.at[pl.ds(off, _CHUNK_ROWS)],
                sems_out.at[b])
        else:
            b = (i // 2) % _NBUF_SH
            c = pltpu.make_async_copy(
                shbufs.at[sid, b], out_hbm.at[pl.ds(off, _CHUNK_ROWS)],
                sems_shout.at[b])
        c.start()
        out_copies[i] = c

    waited = [False] * n_chunks
    for i in range(n_chunks + 1):
        if i < n_chunks:
            j = i - 2 * (_NBUF if i % 2 == 0 else _NBUF_SH)
            if j >= 0:
                out_copies[j].wait()
                waited[j] = True
            start_in(i)
        if i >= 1:
            in_copies[i - 1].wait()
            start_out(i - 1)
    for i in range(n_chunks):
        if not waited[i]:
            out_copies[i].wait()


def kernel(x, emb):
    seq, dim = x.shape[1], emb.shape[1]
    mesh = plsc.VectorSubcoreMesh(core_axis_name="c", subcore_axis_name="s")
    k = pl.kernel(
        _sc_copy_body,
        out_type=jax.ShapeDtypeStruct((seq, dim), emb.dtype),
        mesh=mesh,
        scratch_types=[
            pltpu.VMEM((_NBUF, _CHUNK_ROWS, dim), emb.dtype),
            pltpu.VMEM_SHARED((_NS, _NBUF_SH, _CHUNK_ROWS, dim), emb.dtype),
            pltpu.SemaphoreType.DMA((_NBUF,)),
            pltpu.SemaphoreType.DMA((_NBUF,)),
            pltpu.SemaphoreType.DMA((_NBUF_SH,)),
            pltpu.SemaphoreType.DMA((_NBUF_SH,)),
        ],
    )
    return k(emb)
